# baseline (speedup 1.0000x reference)
import functools

import jax
import jax.numpy as jnp
import numpy as np
from jax import lax
from jax.experimental import pallas as pl
from jax.experimental.pallas import tpu as pltpu
from jax.experimental.pallas import tpu_sc as plsc

VOCAB = 100000
HIDDEN = 1024
BATCH = 4
SEQ = 2048

NC = 2
NS = 16
NW = NC * NS
TOTAL = BATCH * SEQ
ROWS_PER_W = TOTAL // NW
CHUNK = 16
NCHUNK = ROWS_PER_W // CHUNK
LANES = 16


def _pos_encoding() -> np.ndarray:
    pos = np.arange(SEQ)[:, None].astype(np.float64)
    i = np.arange(HIDDEN // 2)[None, :].astype(np.float64)
    angle = pos / np.power(10000.0, 2.0 * i / HIDDEN)
    pe = np.zeros((SEQ, HIDDEN), dtype=np.float64)
    pe[:, 0::2] = np.sin(angle)
    pe[:, 1::2] = np.cos(angle)
    return pe.astype(np.float32)


_PE = _pos_encoding()


def _embed_body(ids_hbm, pe_hbm, table_hbm, out_hbm,
                idx_v, buf0, buf1, pe0, pe1,
                g0, g1, p0, p1, o0, o1):
    c = lax.axis_index("c")
    s = lax.axis_index("s")
    wid = s * NC + c
    base = wid * ROWS_PER_W
    s_base = lax.rem(base, SEQ)

    bufs = (buf0, buf1)
    pes = (pe0, pe1)
    gsems = (g0, g1)
    psems = (p0, p1)
    osems = (o0, o1)

    pltpu.sync_copy(ids_hbm.at[wid], idx_v)

    gather_d = [None, None]
    pe_d = [None, None]
    out_d = [None, None]

    def start_fetch(ch):
        k = ch % 2
        gather_d[k] = pltpu.async_copy(
            table_hbm.at[idx_v.at[ch]], bufs[k], gsems[k])
        pe_d[k] = pltpu.async_copy(
            pe_hbm.at[pl.ds(s_base + ch * CHUNK, CHUNK)], pes[k], psems[k])

    start_fetch(0)
    for ch in range(NCHUNK):
        k = ch % 2
        n = (ch + 1) % 2
        if ch + 1 < NCHUNK:
            if out_d[n] is not None:
                out_d[n].wait()
            start_fetch(ch + 1)
        gather_d[k].wait()
        pe_d[k].wait()

        buf = bufs[k]
        pe_v = pes[k]

        def add_row(r, _):
            for j in range(HIDDEN // LANES):
                sl = pl.ds(j * LANES, LANES)
                plsc.addupdate(buf.at[r, sl], pe_v[r, sl])
            return 0

        lax.fori_loop(0, CHUNK, add_row, 0)

        out_d[k] = pltpu.async_copy(
            buf, out_hbm.at[pl.ds(base + ch * CHUNK, CHUNK)], osems[k])

    out_d[0].wait()
    out_d[1].wait()


@jax.jit
def _embed(ids3, pe, table):
    mesh = plsc.VectorSubcoreMesh(core_axis_name="c", subcore_axis_name="s")
    f = pl.kernel(
        _embed_body,
        out_type=jax.ShapeDtypeStruct((TOTAL, HIDDEN), jnp.float32),
        mesh=mesh,
        scratch_types=[
            pltpu.VMEM((NCHUNK, CHUNK), jnp.int32),
            pltpu.VMEM((CHUNK, HIDDEN), jnp.float32),
            pltpu.VMEM((CHUNK, HIDDEN), jnp.float32),
            pltpu.VMEM((CHUNK, HIDDEN), jnp.float32),
            pltpu.VMEM((CHUNK, HIDDEN), jnp.float32),
            pltpu.SemaphoreType.DMA,
            pltpu.SemaphoreType.DMA,
            pltpu.SemaphoreType.DMA,
            pltpu.SemaphoreType.DMA,
            pltpu.SemaphoreType.DMA,
            pltpu.SemaphoreType.DMA,
        ],
    )
    return f(ids3, pe, table)


def kernel(input_ids, token_embed_weight):
    ids3 = input_ids.astype(jnp.int32).reshape(NW, NCHUNK, CHUNK)
    pe = jnp.asarray(_PE)
    out = _embed(ids3, pe, token_embed_weight)
    return out.reshape(BATCH, SEQ, HIDDEN)
